# Initial kernel scaffold; baseline (speedup 1.0000x reference)
import jax
import jax.numpy as jnp
from jax.experimental import pallas as pl


def kernel(preds, targets):
    raise NotImplementedError("write your pallas kernel here")


# speedup vs baseline: 2.9396x; 2.9396x over previous
import functools

import jax
import jax.numpy as jnp
from jax.experimental import pallas as pl

_B = 128
_C = 32768
_ROWS_PER_BLOCK = 8
_K = 5
_EXTRA_W = 2.0


def _wmse_kernel(p_ref, t_ref, acc_ref):
    i = pl.program_id(0)
    p = p_ref[...]
    t = t_ref[...]
    d = p - t
    d2 = d * d
    total = jnp.sum(d2)

    col = jax.lax.broadcasted_iota(jnp.int32, t.shape, 1)
    extra = jnp.float32(0.0)
    for _ in range(_K):
        m = jnp.max(t, axis=1, keepdims=True)
        is_max = t == m
        sel = jnp.min(jnp.where(is_max, col, _C), axis=1, keepdims=True)
        pick = col == sel
        extra = extra + jnp.sum(jnp.where(pick, d2, 0.0))
        t = jnp.where(pick, -jnp.inf, t)

    val = total + _EXTRA_W * extra

    val2d = val.reshape(1, 1)

    @pl.when(i == 0)
    def _init():
        acc_ref[...] = val2d

    @pl.when(i != 0)
    def _acc():
        acc_ref[...] += val2d


@functools.partial(jax.jit, static_argnames=())
def kernel(preds, targets):
    grid = (_B // _ROWS_PER_BLOCK,)
    acc = pl.pallas_call(
        _wmse_kernel,
        grid=grid,
        in_specs=[
            pl.BlockSpec((_ROWS_PER_BLOCK, _C), lambda i: (i, 0)),
            pl.BlockSpec((_ROWS_PER_BLOCK, _C), lambda i: (i, 0)),
        ],
        out_specs=pl.BlockSpec((1, 1), lambda i: (0, 0)),
        out_shape=jax.ShapeDtypeStruct((1, 1), jnp.float32),
    )(preds, targets)
    return (acc[0, 0] / (_B * _C)).astype(jnp.float32)
